# baseline (speedup 1.0000x reference)
import functools

import jax
import jax.numpy as jnp
from jax import lax
from jax.experimental import pallas as pl
from jax.experimental.pallas import tpu as pltpu
from jax.experimental.pallas import tpu_sc as plsc

NC = 2
NS = 16
L = 16


def _tc1_body(x_ref, w_ref, asrc_ref, adst_ref, hw_ref, alpha_ref):
    hw = jnp.dot(x_ref[...], w_ref[...], preferred_element_type=jnp.float32)
    hw_ref[...] = hw
    a_s = jnp.sum(hw * asrc_ref[...], axis=1)
    a_d = jnp.sum(hw * adst_ref[...], axis=1)
    alpha_ref[...] = jnp.stack([a_s, a_d], axis=1)


def _tc2_body(outp_ref, denp_ref, w_ref, asrc_ref, adst_ref, alpha_ref):
    s = outp_ref[0] + outp_ref[1]
    den = denp_ref[:, 0] + denp_ref[:, 1] + jnp.float32(1e-16)
    h = s / den[:, None]
    h1 = jnp.where(h > 0, h, jnp.exp(h) - jnp.float32(1.0))
    hw = jnp.dot(h1, w_ref[...], preferred_element_type=jnp.float32)
    a_s = jnp.sum(hw * asrc_ref[...], axis=1)
    a_d = jnp.sum(hw * adst_ref[...], axis=1)
    alpha_ref[...] = jnp.stack([a_s, a_d], axis=1)


def _sc1_body(srcT, dstT, hw_hbm, alpha_hbm, outp, den0, den1,
              src_v, dst_v, p_v, av, rows_a, rows_b,
              out_sh, denp_sh, sem_a, sem_b, sem_sa, sem_sb, sem_d,
              *, n_pad, npass, rbp):
    cid = lax.axis_index("c")
    sid = lax.axis_index("s")
    wid = sid * NC + cid
    rows_per_tile = n_pad // NS

    def zero_rows(r, _):
        for di in range(8):
            rows_a[r, pl.ds(di * L, L)] = jnp.zeros((L,), jnp.float32)
        return 0
    lax.fori_loop(0, 80, zero_rows, 0)

    @pl.when(sid == 0)
    def _():
        def zero_av(i, _):
            av[pl.ds(i * L, L)] = jnp.zeros((L,), jnp.float32)
            return 0
        lax.fori_loop(0, n_pad // L, zero_av, 0)
        pltpu.sync_copy(av.at[pl.ds(0, n_pad)], denp_sh)

    base = sid * rows_per_tile
    for k in range(rows_per_tile // 80):
        pltpu.sync_copy(rows_a, out_sh.at[pl.ds(base + k * 80, 80)])

    pltpu.sync_copy(alpha_hbm, av)
    plsc.subcore_barrier()

    one = jnp.ones((L,), jnp.int32)

    def scale_chunk(c, rows_v):
        @plsc.parallel_loop(0, 80, step=1, unroll=4)
        def _(r):
            splat = plsc.load_gather(
                p_v,
                [jnp.full((L,), c, jnp.int32), jnp.full((L,), r, jnp.int32)])
            for di in range(8):
                rows_v[r, pl.ds(di * L, L)] = (
                    rows_v[r, pl.ds(di * L, L)] * splat)

    for pass_i in range(npass):
        pltpu.sync_copy(srcT.at[wid, pass_i], src_v)
        pltpu.sync_copy(dstT.at[wid, pass_i], dst_v)

        def p_row(r, _):
            for j in range(5):
                s16 = src_v[r, pl.ds(j * L, L)]
                d16 = dst_v[r, pl.ds(j * L, L)]
                gs = plsc.load_gather(av, [s16 + s16])
                gd = plsc.load_gather(av, [d16 + d16 + one])
                v = gs + gd
                e = jnp.where(v >= 0, v, v * jnp.float32(0.2))
                p_v[r, pl.ds(j * L, L)] = jnp.exp(e)
            return 0
        lax.fori_loop(0, rbp, p_row, 0)

        def den_rows(g, _):
            for u in range(5):
                r = g * 5 + u
                pltpu.async_copy(
                    p_v.at[r], denp_sh.at[dst_v.at[r]], sem_d, add=True)
            for u in range(5):
                r = g * 5 + u
                pltpu.make_async_copy(
                    p_v.at[r], denp_sh.at[dst_v.at[r]], sem_d).wait()
            return 0
        lax.fori_loop(0, rbp // 5, den_rows, 0)

        pltpu.async_copy(hw_hbm.at[src_v.at[0]], rows_a, sem_a)
        pltpu.make_async_copy(hw_hbm.at[src_v.at[0]], rows_a, sem_a).wait()
        pltpu.async_copy(hw_hbm.at[src_v.at[1]], rows_b, sem_b)
        scale_chunk(0, rows_a)
        pltpu.async_copy(rows_a, out_sh.at[dst_v.at[0]], sem_sa, add=True)

        def agg_pair(c2, _):
            c = c2 * 2 + 1
            pltpu.make_async_copy(hw_hbm.at[src_v.at[c]], rows_b, sem_b).wait()
            pltpu.make_async_copy(
                rows_a, out_sh.at[dst_v.at[c - 1]], sem_sa).wait()
            pltpu.async_copy(hw_hbm.at[src_v.at[c + 1]], rows_a, sem_a)
            scale_chunk(c, rows_b)
            pltpu.async_copy(rows_b, out_sh.at[dst_v.at[c]], sem_sb, add=True)

            pltpu.make_async_copy(
                hw_hbm.at[src_v.at[c + 1]], rows_a, sem_a).wait()
            pltpu.make_async_copy(
                rows_b, out_sh.at[dst_v.at[c]], sem_sb).wait()
            pltpu.async_copy(hw_hbm.at[src_v.at[c + 2]], rows_b, sem_b)
            scale_chunk(c + 1, rows_a)
            pltpu.async_copy(
                rows_a, out_sh.at[dst_v.at[c + 1]], sem_sa, add=True)
            return 0
        lax.fori_loop(0, (rbp - 3) // 2, agg_pair, 0)

        c = rbp - 2
        pltpu.make_async_copy(hw_hbm.at[src_v.at[c]], rows_b, sem_b).wait()
        pltpu.make_async_copy(
            rows_a, out_sh.at[dst_v.at[c - 1]], sem_sa).wait()
        pltpu.async_copy(hw_hbm.at[src_v.at[c + 1]], rows_a, sem_a)
        scale_chunk(c, rows_b)
        pltpu.async_copy(rows_b, out_sh.at[dst_v.at[c]], sem_sb, add=True)
        pltpu.make_async_copy(hw_hbm.at[src_v.at[c + 1]], rows_a, sem_a).wait()
        pltpu.make_async_copy(
            rows_b, out_sh.at[dst_v.at[c]], sem_sb).wait()
        scale_chunk(c + 1, rows_a)
        pltpu.sync_copy(rows_a, out_sh.at[dst_v.at[c + 1]], add=True)

    plsc.subcore_barrier()

    pltpu.sync_copy(out_sh.at[pl.ds(base, rows_per_tile)],
                    outp.at[cid, pl.ds(base, rows_per_tile)])

    @pl.when((sid == 0) & (cid == 0))
    def _():
        pltpu.sync_copy(denp_sh, den0)

    @pl.when((sid == 0) & (cid == 1))
    def _():
        pltpu.sync_copy(denp_sh, den1)


def _sc2_body(srcT, dstT, alpha_hbm, attn_out,
              src_v, dst_v, p_v, av, denv, zbuf,
              den2_sh, sem,
              *, n_nodes, rb2):
    cid = lax.axis_index("c")
    sid = lax.axis_index("s")

    def zero_zbuf(i, _):
        zbuf[pl.ds(i * L, L)] = jnp.zeros((L,), jnp.float32)
        return 0
    lax.fori_loop(0, n_nodes // L, zero_zbuf, 0)

    @pl.when(sid == 0)
    def _():
        pltpu.sync_copy(zbuf, den2_sh)

    pltpu.sync_copy(alpha_hbm, av)
    pltpu.sync_copy(srcT.at[sid], src_v)
    pltpu.sync_copy(dstT.at[sid], dst_v)

    plsc.subcore_barrier()

    one = jnp.ones((L,), jnp.int32)

    def p_row(r, _):
        for j in range(5):
            s16 = src_v[r, pl.ds(j * L, L)]
            d16 = dst_v[r, pl.ds(j * L, L)]
            gs = plsc.load_gather(av, [s16 + s16])
            gd = plsc.load_gather(av, [d16 + d16 + one])
            v = gs + gd
            e = jnp.where(v >= 0, v, v * jnp.float32(0.2))
            p_v[r, pl.ds(j * L, L)] = jnp.exp(e)
        return 0
    lax.fori_loop(0, rb2, p_row, 0)

    def den_rows(g, _):
        for u in range(5):
            r = g * 5 + u
            pltpu.async_copy(p_v.at[r], den2_sh.at[dst_v.at[r]], sem, add=True)
        for u in range(5):
            r = g * 5 + u
            pltpu.make_async_copy(p_v.at[r], den2_sh.at[dst_v.at[r]], sem).wait()
        return 0
    lax.fori_loop(0, rb2 // 5, den_rows, 0)

    plsc.subcore_barrier()

    pltpu.sync_copy(den2_sh, denv)

    half = rb2 // NC
    def attn_row(r2, _):
        r = cid * half + r2
        for j in range(5):
            d16 = dst_v[r, pl.ds(j * L, L)]
            den = plsc.load_gather(denv, [d16])
            p = p_v[r, pl.ds(j * L, L)]
            p_v[r, pl.ds(j * L, L)] = p / (den + jnp.float32(1e-16))
        return 0
    lax.fori_loop(0, half, attn_row, 0)

    pltpu.sync_copy(p_v.at[pl.ds(cid * half, half)], attn_out.at[sid, cid])


def kernel(x, edge_index, W1, a_src1, a_dst1, W2, a_src2, a_dst2):
    n, d = x.shape
    e = edge_index.shape[1]
    assert d == 128 and n % NS == 0 and n % L == 0 and e % (NC * NS * 80) == 0

    nw = NC * NS
    ce = e // nw
    rb = ce // 80
    ce2 = e // NS
    rb2 = ce2 // 80

    src = edge_index[0]
    dst = edge_index[1]
    src16 = src.reshape(NS, rb2, 80)
    dst16 = dst.reshape(NS, rb2, 80)

    blk = 1000
    grid = n // blk

    a1s = a_src1.reshape(1, d)
    a1d = a_dst1.reshape(1, d)
    a2s = a_src2.reshape(1, d)
    a2d = a_dst2.reshape(1, d)

    hw1, alpha1 = pl.pallas_call(
        _tc1_body,
        grid=(grid,),
        in_specs=[
            pl.BlockSpec((blk, d), lambda i: (i, 0)),
            pl.BlockSpec((d, d), lambda i: (0, 0)),
            pl.BlockSpec((1, d), lambda i: (0, 0)),
            pl.BlockSpec((1, d), lambda i: (0, 0)),
        ],
        out_specs=[
            pl.BlockSpec((blk, d), lambda i: (i, 0)),
            pl.BlockSpec((blk, 2), lambda i: (i, 0)),
        ],
        out_shape=[
            jax.ShapeDtypeStruct((n, d), jnp.float32),
            jax.ShapeDtypeStruct((n, 2), jnp.float32),
        ],
    )(x, W1, a1s, a1d)

    mesh = plsc.VectorSubcoreMesh(
        core_axis_name="c", subcore_axis_name="s",
        num_cores=NC, num_subcores=NS)

    n_pad = ((n + 1279) // 1280) * 1280

    sc_params = pltpu.CompilerParams(needs_layout_passes=False, use_tc_tiling_on_sc=False)

    npass = 5
    rbp = rb // npass
    sc1 = pl.kernel(
        functools.partial(_sc1_body, n_pad=n_pad, npass=npass, rbp=rbp),
        compiler_params=sc_params,
        out_type=[
            jax.ShapeDtypeStruct((NC, n_pad, d), jnp.float32),
            jax.ShapeDtypeStruct((n_pad,), jnp.float32),
            jax.ShapeDtypeStruct((n_pad,), jnp.float32),
        ],
        mesh=mesh,
        scratch_types=[
            pltpu.VMEM((rbp, 80), jnp.int32),
            pltpu.VMEM((rbp, 80), jnp.int32),
            pltpu.VMEM((rbp, 80), jnp.float32),
            pltpu.VMEM((2 * n,), jnp.float32),
            pltpu.VMEM((80, d), jnp.float32),
            pltpu.VMEM((80, d), jnp.float32),
            pltpu.VMEM_SHARED((n_pad, d), jnp.float32),
            pltpu.VMEM_SHARED((n_pad,), jnp.float32),
            pltpu.SemaphoreType.DMA,
            pltpu.SemaphoreType.DMA,
            pltpu.SemaphoreType.DMA,
            pltpu.SemaphoreType.DMA,
            pltpu.SemaphoreType.DMA,
        ],
    )
    outp, den0, den1 = sc1(src.reshape(nw, npass, rbp, 80),
                           dst.reshape(nw, npass, rbp, 80),
                           hw1, alpha1.reshape(2 * n))
    denp_t = jnp.stack([den0, den1], axis=1)

    blk2 = n_pad // grid
    alpha2 = pl.pallas_call(
        _tc2_body,
        grid=(grid,),
        in_specs=[
            pl.BlockSpec((NC, blk2, d), lambda i: (0, i, 0)),
            pl.BlockSpec((blk2, NC), lambda i: (i, 0)),
            pl.BlockSpec((d, d), lambda i: (0, 0)),
            pl.BlockSpec((1, d), lambda i: (0, 0)),
            pl.BlockSpec((1, d), lambda i: (0, 0)),
        ],
        out_specs=[pl.BlockSpec((blk2, 2), lambda i: (i, 0))],
        out_shape=[jax.ShapeDtypeStruct((n_pad, 2), jnp.float32)],
    )(outp, denp_t, W2, a2s, a2d)[0]

    sc2 = pl.kernel(
        functools.partial(_sc2_body, n_nodes=n, rb2=rb2),
        compiler_params=sc_params,
        out_type=jax.ShapeDtypeStruct((NS, NC, rb2 // NC, 80), jnp.float32),
        mesh=mesh,
        scratch_types=[
            pltpu.VMEM((rb2, 80), jnp.int32),
            pltpu.VMEM((rb2, 80), jnp.int32),
            pltpu.VMEM((rb2, 80), jnp.float32),
            pltpu.VMEM((2 * n_pad,), jnp.float32),
            pltpu.VMEM((n,), jnp.float32),
            pltpu.VMEM((n,), jnp.float32),
            pltpu.VMEM_SHARED((n,), jnp.float32),
            pltpu.SemaphoreType.DMA,
        ],
    )
    attn = sc2(src16, dst16, alpha2.reshape(2 * n_pad))
    return attn.reshape(e)


# speedup vs baseline: 1.0447x; 1.0447x over previous
import functools

import jax
import jax.numpy as jnp
from jax import lax
from jax.experimental import pallas as pl
from jax.experimental.pallas import tpu as pltpu
from jax.experimental.pallas import tpu_sc as plsc

NC = 2
NS = 16
L = 16


def _tc1_body(x_ref, w_ref, asrc_ref, adst_ref, hw_ref, alpha_ref):
    hw = jnp.dot(x_ref[...], w_ref[...], preferred_element_type=jnp.float32)
    hw_ref[...] = hw
    a_s = jnp.sum(hw * asrc_ref[...], axis=1)
    a_d = jnp.sum(hw * adst_ref[...], axis=1)
    alpha_ref[...] = jnp.stack([a_s, a_d], axis=1)


def _tc2_body(outp_ref, denp_ref, w_ref, asrc_ref, adst_ref, alpha_ref):
    s = outp_ref[0] + outp_ref[1]
    den = denp_ref[:, 0] + denp_ref[:, 1] + jnp.float32(1e-16)
    h = s / den[:, None]
    h1 = jnp.where(h > 0, h, jnp.exp(h) - jnp.float32(1.0))
    hw = jnp.dot(h1, w_ref[...], preferred_element_type=jnp.float32)
    a_s = jnp.sum(hw * asrc_ref[...], axis=1)
    a_d = jnp.sum(hw * adst_ref[...], axis=1)
    alpha_ref[...] = jnp.stack([a_s, a_d], axis=1)


def _sc1_body(srcT, dstT, hw_hbm, alpha_hbm, outp, den0, den1,
              src_v, dst_v, p_v, av, rows_a, rows_b,
              out_sh, denp_sh, sem_a, sem_b, sem_sa, sem_sb, sem_d,
              *, n_pad, npass, rbp):
    cid = lax.axis_index("c")
    sid = lax.axis_index("s")
    wid = sid * NC + cid
    rows_per_tile = n_pad // NS

    def zero_rows(r, _):
        for di in range(8):
            rows_a[r, pl.ds(di * L, L)] = jnp.zeros((L,), jnp.float32)
        return 0
    lax.fori_loop(0, 80, zero_rows, 0)

    @pl.when(sid == 0)
    def _():
        def zero_av(i, _):
            av[pl.ds(i * L, L)] = jnp.zeros((L,), jnp.float32)
            return 0
        lax.fori_loop(0, n_pad // L, zero_av, 0)
        pltpu.sync_copy(av.at[pl.ds(0, n_pad)], denp_sh)

    base = sid * rows_per_tile
    for k in range(rows_per_tile // 80):
        pltpu.sync_copy(rows_a, out_sh.at[pl.ds(base + k * 80, 80)])

    pltpu.sync_copy(alpha_hbm, av)
    plsc.subcore_barrier()

    one = jnp.ones((L,), jnp.int32)

    def scale_chunk(c, rows_v):
        @plsc.parallel_loop(0, 80, step=1, unroll=4)
        def _(r):
            splat = plsc.load_gather(
                p_v,
                [jnp.full((L,), c, jnp.int32), jnp.full((L,), r, jnp.int32)])
            for di in range(8):
                rows_v[r, pl.ds(di * L, L)] = (
                    rows_v[r, pl.ds(di * L, L)] * splat)

    for pass_i in range(npass):
        pltpu.sync_copy(srcT.at[wid, pass_i], src_v)
        pltpu.sync_copy(dstT.at[wid, pass_i], dst_v)

        def p_row(r, _):
            for j in range(5):
                s16 = src_v[r, pl.ds(j * L, L)]
                d16 = dst_v[r, pl.ds(j * L, L)]
                gs = plsc.load_gather(av, [s16 + s16])
                gd = plsc.load_gather(av, [d16 + d16 + one])
                v = gs + gd
                e = jnp.where(v >= 0, v, v * jnp.float32(0.2))
                p_v[r, pl.ds(j * L, L)] = jnp.exp(e)
            return 0
        lax.fori_loop(0, rbp, p_row, 0)

        def den_rows(g, _):
            for u in range(5):
                r = g * 5 + u
                pltpu.async_copy(
                    p_v.at[r], denp_sh.at[dst_v.at[r]], sem_d, add=True)
            for u in range(5):
                r = g * 5 + u
                pltpu.make_async_copy(
                    p_v.at[r], denp_sh.at[dst_v.at[r]], sem_d).wait()
            return 0
        lax.fori_loop(0, rbp // 5, den_rows, 0)

        pltpu.async_copy(hw_hbm.at[src_v.at[0]], rows_a, sem_a)
        pltpu.make_async_copy(hw_hbm.at[src_v.at[0]], rows_a, sem_a).wait()
        pltpu.async_copy(hw_hbm.at[src_v.at[1]], rows_b, sem_b)
        scale_chunk(0, rows_a)
        pltpu.async_copy(rows_a, out_sh.at[dst_v.at[0]], sem_sa, add=True)

        def agg_pair(c2, _):
            c = c2 * 2 + 1
            pltpu.make_async_copy(
                rows_a, out_sh.at[dst_v.at[c - 1]], sem_sa).wait()
            pltpu.async_copy(hw_hbm.at[src_v.at[c + 1]], rows_a, sem_a)
            pltpu.make_async_copy(hw_hbm.at[src_v.at[c]], rows_b, sem_b).wait()
            scale_chunk(c, rows_b)
            pltpu.async_copy(rows_b, out_sh.at[dst_v.at[c]], sem_sb, add=True)

            pltpu.make_async_copy(
                rows_b, out_sh.at[dst_v.at[c]], sem_sb).wait()
            pltpu.async_copy(hw_hbm.at[src_v.at[c + 2]], rows_b, sem_b)
            pltpu.make_async_copy(
                hw_hbm.at[src_v.at[c + 1]], rows_a, sem_a).wait()
            scale_chunk(c + 1, rows_a)
            pltpu.async_copy(
                rows_a, out_sh.at[dst_v.at[c + 1]], sem_sa, add=True)
            return 0
        lax.fori_loop(0, (rbp - 3) // 2, agg_pair, 0)

        c = rbp - 2
        pltpu.make_async_copy(hw_hbm.at[src_v.at[c]], rows_b, sem_b).wait()
        pltpu.make_async_copy(
            rows_a, out_sh.at[dst_v.at[c - 1]], sem_sa).wait()
        pltpu.async_copy(hw_hbm.at[src_v.at[c + 1]], rows_a, sem_a)
        scale_chunk(c, rows_b)
        pltpu.async_copy(rows_b, out_sh.at[dst_v.at[c]], sem_sb, add=True)
        pltpu.make_async_copy(hw_hbm.at[src_v.at[c + 1]], rows_a, sem_a).wait()
        pltpu.make_async_copy(
            rows_b, out_sh.at[dst_v.at[c]], sem_sb).wait()
        scale_chunk(c + 1, rows_a)
        pltpu.sync_copy(rows_a, out_sh.at[dst_v.at[c + 1]], add=True)

    plsc.subcore_barrier()

    pltpu.sync_copy(out_sh.at[pl.ds(base, rows_per_tile)],
                    outp.at[cid, pl.ds(base, rows_per_tile)])

    @pl.when((sid == 0) & (cid == 0))
    def _():
        pltpu.sync_copy(denp_sh, den0)

    @pl.when((sid == 0) & (cid == 1))
    def _():
        pltpu.sync_copy(denp_sh, den1)


def _sc2_body(srcT, dstT, alpha_hbm, attn_out,
              src_v, dst_v, p_v, av, denv, zbuf,
              den2_sh, sem,
              *, n_nodes, rb2):
    cid = lax.axis_index("c")
    sid = lax.axis_index("s")

    def zero_zbuf(i, _):
        zbuf[pl.ds(i * L, L)] = jnp.zeros((L,), jnp.float32)
        return 0
    lax.fori_loop(0, n_nodes // L, zero_zbuf, 0)

    @pl.when(sid == 0)
    def _():
        pltpu.sync_copy(zbuf, den2_sh)

    pltpu.sync_copy(alpha_hbm, av)
    pltpu.sync_copy(srcT.at[sid], src_v)
    pltpu.sync_copy(dstT.at[sid], dst_v)

    plsc.subcore_barrier()

    one = jnp.ones((L,), jnp.int32)

    def p_row(r, _):
        for j in range(5):
            s16 = src_v[r, pl.ds(j * L, L)]
            d16 = dst_v[r, pl.ds(j * L, L)]
            gs = plsc.load_gather(av, [s16 + s16])
            gd = plsc.load_gather(av, [d16 + d16 + one])
            v = gs + gd
            e = jnp.where(v >= 0, v, v * jnp.float32(0.2))
            p_v[r, pl.ds(j * L, L)] = jnp.exp(e)
        return 0
    lax.fori_loop(0, rb2, p_row, 0)

    def den_rows(g, _):
        for u in range(5):
            r = g * 5 + u
            pltpu.async_copy(p_v.at[r], den2_sh.at[dst_v.at[r]], sem, add=True)
        for u in range(5):
            r = g * 5 + u
            pltpu.make_async_copy(p_v.at[r], den2_sh.at[dst_v.at[r]], sem).wait()
        return 0
    lax.fori_loop(0, rb2 // 5, den_rows, 0)

    plsc.subcore_barrier()

    pltpu.sync_copy(den2_sh, denv)

    half = rb2 // NC
    def attn_row(r2, _):
        r = cid * half + r2
        for j in range(5):
            d16 = dst_v[r, pl.ds(j * L, L)]
            den = plsc.load_gather(denv, [d16])
            p = p_v[r, pl.ds(j * L, L)]
            p_v[r, pl.ds(j * L, L)] = p / (den + jnp.float32(1e-16))
        return 0
    lax.fori_loop(0, half, attn_row, 0)

    pltpu.sync_copy(p_v.at[pl.ds(cid * half, half)], attn_out.at[sid, cid])


def kernel(x, edge_index, W1, a_src1, a_dst1, W2, a_src2, a_dst2):
    n, d = x.shape
    e = edge_index.shape[1]
    assert d == 128 and n % NS == 0 and n % L == 0 and e % (NC * NS * 80) == 0

    nw = NC * NS
    ce = e // nw
    rb = ce // 80
    ce2 = e // NS
    rb2 = ce2 // 80

    src = edge_index[0]
    dst = edge_index[1]
    src16 = src.reshape(NS, rb2, 80)
    dst16 = dst.reshape(NS, rb2, 80)

    blk = 1000
    grid = n // blk

    a1s = a_src1.reshape(1, d)
    a1d = a_dst1.reshape(1, d)
    a2s = a_src2.reshape(1, d)
    a2d = a_dst2.reshape(1, d)

    hw1, alpha1 = pl.pallas_call(
        _tc1_body,
        grid=(grid,),
        in_specs=[
            pl.BlockSpec((blk, d), lambda i: (i, 0)),
            pl.BlockSpec((d, d), lambda i: (0, 0)),
            pl.BlockSpec((1, d), lambda i: (0, 0)),
            pl.BlockSpec((1, d), lambda i: (0, 0)),
        ],
        out_specs=[
            pl.BlockSpec((blk, d), lambda i: (i, 0)),
            pl.BlockSpec((blk, 2), lambda i: (i, 0)),
        ],
        out_shape=[
            jax.ShapeDtypeStruct((n, d), jnp.float32),
            jax.ShapeDtypeStruct((n, 2), jnp.float32),
        ],
    )(x, W1, a1s, a1d)

    mesh = plsc.VectorSubcoreMesh(
        core_axis_name="c", subcore_axis_name="s",
        num_cores=NC, num_subcores=NS)

    n_pad = ((n + 1279) // 1280) * 1280

    sc_params = pltpu.CompilerParams(needs_layout_passes=False, use_tc_tiling_on_sc=False)

    npass = 5
    rbp = rb // npass
    sc1 = pl.kernel(
        functools.partial(_sc1_body, n_pad=n_pad, npass=npass, rbp=rbp),
        compiler_params=sc_params,
        out_type=[
            jax.ShapeDtypeStruct((NC, n_pad, d), jnp.float32),
            jax.ShapeDtypeStruct((n_pad,), jnp.float32),
            jax.ShapeDtypeStruct((n_pad,), jnp.float32),
        ],
        mesh=mesh,
        scratch_types=[
            pltpu.VMEM((rbp, 80), jnp.int32),
            pltpu.VMEM((rbp, 80), jnp.int32),
            pltpu.VMEM((rbp, 80), jnp.float32),
            pltpu.VMEM((2 * n,), jnp.float32),
            pltpu.VMEM((80, d), jnp.float32),
            pltpu.VMEM((80, d), jnp.float32),
            pltpu.VMEM_SHARED((n_pad, d), jnp.float32),
            pltpu.VMEM_SHARED((n_pad,), jnp.float32),
            pltpu.SemaphoreType.DMA,
            pltpu.SemaphoreType.DMA,
            pltpu.SemaphoreType.DMA,
            pltpu.SemaphoreType.DMA,
            pltpu.SemaphoreType.DMA,
        ],
    )
    outp, den0, den1 = sc1(src.reshape(nw, npass, rbp, 80),
                           dst.reshape(nw, npass, rbp, 80),
                           hw1, alpha1.reshape(2 * n))
    denp_t = jnp.stack([den0, den1], axis=1)

    blk2 = n_pad // grid
    alpha2 = pl.pallas_call(
        _tc2_body,
        grid=(grid,),
        in_specs=[
            pl.BlockSpec((NC, blk2, d), lambda i: (0, i, 0)),
            pl.BlockSpec((blk2, NC), lambda i: (i, 0)),
            pl.BlockSpec((d, d), lambda i: (0, 0)),
            pl.BlockSpec((1, d), lambda i: (0, 0)),
            pl.BlockSpec((1, d), lambda i: (0, 0)),
        ],
        out_specs=[pl.BlockSpec((blk2, 2), lambda i: (i, 0))],
        out_shape=[jax.ShapeDtypeStruct((n_pad, 2), jnp.float32)],
    )(outp, denp_t, W2, a2s, a2d)[0]

    sc2 = pl.kernel(
        functools.partial(_sc2_body, n_nodes=n, rb2=rb2),
        compiler_params=sc_params,
        out_type=jax.ShapeDtypeStruct((NS, NC, rb2 // NC, 80), jnp.float32),
        mesh=mesh,
        scratch_types=[
            pltpu.VMEM((rb2, 80), jnp.int32),
            pltpu.VMEM((rb2, 80), jnp.int32),
            pltpu.VMEM((rb2, 80), jnp.float32),
            pltpu.VMEM((2 * n_pad,), jnp.float32),
            pltpu.VMEM((n,), jnp.float32),
            pltpu.VMEM((n,), jnp.float32),
            pltpu.VMEM_SHARED((n,), jnp.float32),
            pltpu.SemaphoreType.DMA,
        ],
    )
    attn = sc2(src16, dst16, alpha2.reshape(2 * n_pad))
    return attn.reshape(e)
